# Initial kernel scaffold; baseline (speedup 1.0000x reference)
import jax
import jax.numpy as jnp
from jax.experimental import pallas as pl


def kernel(inputs, first_a, first_t, Awij, Awij2):
    raise NotImplementedError("write your pallas kernel here")


# speedup vs baseline: 165.6320x; 165.6320x over previous
import functools

import jax
import jax.numpy as jnp
from jax.experimental import pallas as pl
from jax.experimental.pallas import tpu as pltpu

_NA, _NT = 2048, 2048
_C = 4
_STEPS = 2
_BJ = 256
_NB = _NA // _BJ


def _body(s2_ref, s1_ref, fa_ref, ft_ref, x_ref, out_a_ref, out_t_ref,
          ua_ref, ut_ref):
    p = pl.program_id(0)
    i = pl.program_id(1)

    @pl.when(jnp.logical_and(p == 0, i == 0))
    def _init():
        ua_ref[...] = fa_ref[...]
        ut_ref[...] = ft_ref[...]

    x = x_ref[...]
    m0 = (x == 0).astype(jnp.float32)
    m1 = (x == 1).astype(jnp.float32)
    m2 = (x == 2).astype(jnp.float32)

    @pl.when(p % 2 == 0)
    def _phase_a():
        s2 = s2_ref[...]
        u = ut_ref[...]
        r3 = s2[3:4, :]
        msg = jax.lax.dot_general(
            m0, u * (s2[0:1, :] - r3), (((1,), (0,)), ((), ())),
            preferred_element_type=jnp.float32,
            precision=jax.lax.Precision.HIGHEST)
        msg += jax.lax.dot_general(
            m1, u * (s2[1:2, :] - r3), (((1,), (0,)), ((), ())),
            preferred_element_type=jnp.float32,
            precision=jax.lax.Precision.HIGHEST)
        msg += jax.lax.dot_general(
            m2, u * (s2[2:3, :] - r3), (((1,), (0,)), ((), ())),
            preferred_element_type=jnp.float32,
            precision=jax.lax.Precision.HIGHEST)
        msg += jnp.sum(u * r3, axis=0, keepdims=True)
        ua_ref[pl.ds(i * _BJ, _BJ), :] += msg

    @pl.when(p % 2 == 1)
    def _phase_t():
        s1 = s1_ref[...]
        ua = ua_ref[pl.ds(i * _BJ, _BJ), :]
        r3 = s1[3:4, :]
        part = jax.lax.dot_general(
            m0, ua * (s1[0:1, :] - r3), (((0,), (0,)), ((), ())),
            preferred_element_type=jnp.float32,
            precision=jax.lax.Precision.HIGHEST)
        part += jax.lax.dot_general(
            m1, ua * (s1[1:2, :] - r3), (((0,), (0,)), ((), ())),
            preferred_element_type=jnp.float32,
            precision=jax.lax.Precision.HIGHEST)
        part += jax.lax.dot_general(
            m2, ua * (s1[2:3, :] - r3), (((0,), (0,)), ((), ())),
            preferred_element_type=jnp.float32,
            precision=jax.lax.Precision.HIGHEST)
        part += jnp.sum(ua * r3, axis=0, keepdims=True)
        ut_ref[...] += part

    @pl.when(jnp.logical_and(p == 2 * _STEPS - 1, i == _NB - 1))
    def _finish():
        out_a_ref[...] = ua_ref[...]
        out_t_ref[...] = ut_ref[...]


@functools.partial(jax.jit, static_argnames=())
def _mpnn(inputs, first_a, first_t, s2, s1):
    out_a, out_t = pl.pallas_call(
        _body,
        grid=(2 * _STEPS, _NB),
        in_specs=[
            pl.BlockSpec((_C, _C), lambda p, i: (0, 0)),
            pl.BlockSpec((_C, _C), lambda p, i: (0, 0)),
            pl.BlockSpec((_NA, _C), lambda p, i: (0, 0)),
            pl.BlockSpec((_NT, _C), lambda p, i: (0, 0)),
            pl.BlockSpec((_BJ, _NT), lambda p, i: (i, 0)),
        ],
        out_specs=[
            pl.BlockSpec((_NA, _C), lambda p, i: (0, 0)),
            pl.BlockSpec((_NT, _C), lambda p, i: (0, 0)),
        ],
        out_shape=[
            jax.ShapeDtypeStruct((_NA, _C), jnp.float32),
            jax.ShapeDtypeStruct((_NT, _C), jnp.float32),
        ],
        scratch_shapes=[
            pltpu.VMEM((_NA, _C), jnp.float32),
            pltpu.VMEM((_NT, _C), jnp.float32),
        ],
        compiler_params=pltpu.CompilerParams(
            dimension_semantics=("arbitrary", "arbitrary")),
    )(s2, s1, first_a, first_t, inputs)
    return out_a, out_t


def kernel(inputs, first_a, first_t, Awij, Awij2):
    s2 = jnp.sum(Awij2, axis=1)
    s1 = jnp.sum(Awij, axis=1)
    out_a, out_t = _mpnn(inputs, first_a, first_t, s2, s1)
    pad = jnp.zeros((_NA, _C), dtype=out_a.dtype)
    top = jnp.concatenate([out_a, pad], axis=1)
    bot = jnp.concatenate([out_t, pad], axis=1)
    return jnp.concatenate([top, bot], axis=0)


# speedup vs baseline: 333.3793x; 2.0128x over previous
import functools

import jax
import jax.numpy as jnp
from jax.experimental import pallas as pl
from jax.experimental.pallas import tpu as pltpu

_NA, _NT = 2048, 2048
_C = 4
_STEPS = 2
_BJ = 256
_NB = _NA // _BJ


def _body(s2_ref, s1_ref, fa_ref, ft_ref, x_ref, out_a_ref, out_t_ref,
          ua_ref, ut_ref):
    p = pl.program_id(0)
    i = pl.program_id(1)

    @pl.when(jnp.logical_and(p == 0, i == 0))
    def _init():
        ua_ref[...] = fa_ref[...]
        ut_ref[...] = ft_ref[...]

    x = x_ref[...]
    ms = [(x == e).astype(jnp.bfloat16) for e in range(3)]

    def _masked_sum(s_ref, u, dims):
        s = s_ref[...]
        r3 = s[3:4, :]
        acc = jnp.sum(u * r3, axis=0, keepdims=True)
        for e in range(3):
            us = u * (s[e:e + 1, :] - r3)
            uh = us.astype(jnp.bfloat16)
            ul = (us - uh.astype(jnp.float32)).astype(jnp.bfloat16)
            acc += jax.lax.dot_general(
                ms[e], uh, dims, preferred_element_type=jnp.float32)
            acc += jax.lax.dot_general(
                ms[e], ul, dims, preferred_element_type=jnp.float32)
        return acc

    @pl.when(p % 2 == 0)
    def _phase_a():
        u = ut_ref[...]
        msg = _masked_sum(s2_ref, u, (((1,), (0,)), ((), ())))
        ua_ref[pl.ds(i * _BJ, _BJ), :] += msg

    @pl.when(p % 2 == 1)
    def _phase_t():
        ua = ua_ref[pl.ds(i * _BJ, _BJ), :]
        part = _masked_sum(s1_ref, ua, (((0,), (0,)), ((), ())))
        ut_ref[...] += part

    @pl.when(jnp.logical_and(p == 2 * _STEPS - 1, i == _NB - 1))
    def _finish():
        out_a_ref[...] = ua_ref[...]
        out_t_ref[...] = ut_ref[...]


@functools.partial(jax.jit, static_argnames=())
def _mpnn(inputs, first_a, first_t, s2, s1):
    out_a, out_t = pl.pallas_call(
        _body,
        grid=(2 * _STEPS, _NB),
        in_specs=[
            pl.BlockSpec((_C, _C), lambda p, i: (0, 0)),
            pl.BlockSpec((_C, _C), lambda p, i: (0, 0)),
            pl.BlockSpec((_NA, _C), lambda p, i: (0, 0)),
            pl.BlockSpec((_NT, _C), lambda p, i: (0, 0)),
            pl.BlockSpec((_BJ, _NT), lambda p, i: (i, 0)),
        ],
        out_specs=[
            pl.BlockSpec((_NA, _C), lambda p, i: (0, 0)),
            pl.BlockSpec((_NT, _C), lambda p, i: (0, 0)),
        ],
        out_shape=[
            jax.ShapeDtypeStruct((_NA, _C), jnp.float32),
            jax.ShapeDtypeStruct((_NT, _C), jnp.float32),
        ],
        scratch_shapes=[
            pltpu.VMEM((_NA, _C), jnp.float32),
            pltpu.VMEM((_NT, _C), jnp.float32),
        ],
        compiler_params=pltpu.CompilerParams(
            dimension_semantics=("arbitrary", "arbitrary")),
    )(s2, s1, first_a, first_t, inputs)
    return out_a, out_t


def kernel(inputs, first_a, first_t, Awij, Awij2):
    s2 = jnp.sum(Awij2, axis=1)
    s1 = jnp.sum(Awij, axis=1)
    out_a, out_t = _mpnn(inputs, first_a, first_t, s2, s1)
    pad = jnp.zeros((_NA, _C), dtype=out_a.dtype)
    top = jnp.concatenate([out_a, pad], axis=1)
    bot = jnp.concatenate([out_t, pad], axis=1)
    return jnp.concatenate([top, bot], axis=0)


# speedup vs baseline: 443.2454x; 1.3296x over previous
import functools

import jax
import jax.numpy as jnp
from jax.experimental import pallas as pl
from jax.experimental.pallas import tpu as pltpu

_NA, _NT = 2048, 2048
_C = 4
_STEPS = 2
_BJ = 256
_NB = _NA // _BJ


def _body(s2_ref, s1_ref, fa_ref, ft_ref, x_ref, out_a_ref, out_t_ref,
          ua_ref, ut_ref, m0_ref, m1_ref, m2_ref):
    p = pl.program_id(0)
    i = pl.program_id(1)
    rows = pl.ds(i * _BJ, _BJ)

    @pl.when(jnp.logical_and(p == 0, i == 0))
    def _init():
        ua_ref[...] = fa_ref[...]
        ut_ref[...] = ft_ref[...]

    @pl.when(p == 0)
    def _build_masks():
        x = x_ref[...]
        m0_ref[rows, :] = (x == 0).astype(jnp.bfloat16)
        m1_ref[rows, :] = (x == 1).astype(jnp.bfloat16)
        m2_ref[rows, :] = (x == 2).astype(jnp.bfloat16)

    def _masked_sum(s_ref, u, dims):
        s = s_ref[...]
        r3 = s[3:4, :]
        acc = jnp.sum(u * r3, axis=0, keepdims=True)
        for e, m_ref in enumerate((m0_ref, m1_ref, m2_ref)):
            us = u * (s[e:e + 1, :] - r3)
            uh = us.astype(jnp.bfloat16)
            ul = (us - uh.astype(jnp.float32)).astype(jnp.bfloat16)
            rhs = jnp.concatenate([uh, ul], axis=1)
            both = jax.lax.dot_general(
                m_ref[rows, :], rhs, dims,
                preferred_element_type=jnp.float32)
            acc += both[:, :_C] + both[:, _C:]
        return acc

    @pl.when(p % 2 == 0)
    def _phase_a():
        u = ut_ref[...]
        msg = _masked_sum(s2_ref, u, (((1,), (0,)), ((), ())))
        ua_ref[rows, :] += msg

    @pl.when(p % 2 == 1)
    def _phase_t():
        ua = ua_ref[rows, :]
        part = _masked_sum(s1_ref, ua, (((0,), (0,)), ((), ())))
        ut_ref[...] += part

    @pl.when(jnp.logical_and(p == 2 * _STEPS - 1, i == _NB - 1))
    def _finish():
        out_a_ref[...] = ua_ref[...]
        out_t_ref[...] = ut_ref[...]


@functools.partial(jax.jit, static_argnames=())
def _mpnn(inputs, first_a, first_t, s2, s1):
    out_a, out_t = pl.pallas_call(
        _body,
        grid=(2 * _STEPS, _NB),
        in_specs=[
            pl.BlockSpec((_C, _C), lambda p, i: (0, 0)),
            pl.BlockSpec((_C, _C), lambda p, i: (0, 0)),
            pl.BlockSpec((_NA, _C), lambda p, i: (0, 0)),
            pl.BlockSpec((_NT, _C), lambda p, i: (0, 0)),
            pl.BlockSpec((_BJ, _NT),
                         lambda p, i: (jnp.where(p == 0, i, 0), 0)),
        ],
        out_specs=[
            pl.BlockSpec((_NA, _C), lambda p, i: (0, 0)),
            pl.BlockSpec((_NT, _C), lambda p, i: (0, 0)),
        ],
        out_shape=[
            jax.ShapeDtypeStruct((_NA, _C), jnp.float32),
            jax.ShapeDtypeStruct((_NT, _C), jnp.float32),
        ],
        scratch_shapes=[
            pltpu.VMEM((_NA, _C), jnp.float32),
            pltpu.VMEM((_NT, _C), jnp.float32),
            pltpu.VMEM((_NA, _NT), jnp.bfloat16),
            pltpu.VMEM((_NA, _NT), jnp.bfloat16),
            pltpu.VMEM((_NA, _NT), jnp.bfloat16),
        ],
        compiler_params=pltpu.CompilerParams(
            dimension_semantics=("arbitrary", "arbitrary")),
    )(s2, s1, first_a, first_t, inputs)
    return out_a, out_t


def kernel(inputs, first_a, first_t, Awij, Awij2):
    s2 = jnp.sum(Awij2, axis=1)
    s1 = jnp.sum(Awij, axis=1)
    out_a, out_t = _mpnn(inputs, first_a, first_t, s2, s1)
    pad = jnp.zeros((_NA, _C), dtype=out_a.dtype)
    top = jnp.concatenate([out_a, pad], axis=1)
    bot = jnp.concatenate([out_t, pad], axis=1)
    return jnp.concatenate([top, bot], axis=0)


# speedup vs baseline: 629.4610x; 1.4201x over previous
import functools

import jax
import jax.numpy as jnp
from jax.experimental import pallas as pl
from jax.experimental.pallas import tpu as pltpu

_NA, _NT = 2048, 2048
_C = 4
_STEPS = 2
_BJ = 512
_NB = _NA // _BJ


def _split_hilo(us):
    uh = us.astype(jnp.bfloat16)
    ul = (us - uh.astype(jnp.float32)).astype(jnp.bfloat16)
    return jnp.concatenate([uh, ul], axis=1)


def _body(s2_ref, s1_ref, fa_ref, ft_ref, x_ref, out_a_ref, out_t_ref,
          ua_ref, ut_ref, m0_ref, m1_ref, m2_ref, rhs_ref, ones_ref):
    p = pl.program_id(0)
    i = pl.program_id(1)
    rows = pl.ds(i * _BJ, _BJ)

    @pl.when(jnp.logical_and(p == 0, i == 0))
    def _init():
        ua_ref[...] = fa_ref[...]
        ut_ref[...] = ft_ref[...]

    @pl.when(p == 0)
    def _build_masks():
        x = x_ref[...]
        m0_ref[rows, :] = (x == 0).astype(jnp.bfloat16)
        m1_ref[rows, :] = (x == 1).astype(jnp.bfloat16)
        m2_ref[rows, :] = (x == 2).astype(jnp.bfloat16)

    @pl.when(jnp.logical_and(p % 2 == 0, i == 0))
    def _rhs_a():
        s2 = s2_ref[...]
        r3 = s2[3:4, :]
        u = ut_ref[...]
        for e in range(3):
            rhs_ref[pl.ds(e * _NT, _NT), :] = _split_hilo(
                u * (s2[e:e + 1, :] - r3))
        ones_ref[...] = jnp.sum(u * r3, axis=0, keepdims=True)

    @pl.when(jnp.logical_and(p % 2 == 1, i == 0))
    def _rhs_t():
        s1 = s1_ref[...]
        r3 = s1[3:4, :]
        ua = ua_ref[...]
        for e in range(3):
            rhs_ref[pl.ds(e * _NA, _NA), :] = _split_hilo(
                ua * (s1[e:e + 1, :] - r3))
        ones_ref[...] = jnp.sum(ua * r3, axis=0, keepdims=True)

    @pl.when(p % 2 == 0)
    def _phase_a():
        acc = jax.lax.dot_general(
            m0_ref[rows, :], rhs_ref[pl.ds(0, _NT), :],
            (((1,), (0,)), ((), ())), preferred_element_type=jnp.float32)
        acc += jax.lax.dot_general(
            m1_ref[rows, :], rhs_ref[pl.ds(_NT, _NT), :],
            (((1,), (0,)), ((), ())), preferred_element_type=jnp.float32)
        acc += jax.lax.dot_general(
            m2_ref[rows, :], rhs_ref[pl.ds(2 * _NT, _NT), :],
            (((1,), (0,)), ((), ())), preferred_element_type=jnp.float32)
        msg = acc[:, :_C] + acc[:, _C:] + ones_ref[...]
        ua_ref[rows, :] += msg

    @pl.when(p % 2 == 1)
    def _phase_t():
        rr = pl.ds(i * _BJ, _BJ)
        acc = jax.lax.dot_general(
            m0_ref[rows, :], rhs_ref[rr, :],
            (((0,), (0,)), ((), ())), preferred_element_type=jnp.float32)
        acc += jax.lax.dot_general(
            m1_ref[rows, :], rhs_ref[pl.ds(_NA + i * _BJ, _BJ), :],
            (((0,), (0,)), ((), ())), preferred_element_type=jnp.float32)
        acc += jax.lax.dot_general(
            m2_ref[rows, :], rhs_ref[pl.ds(2 * _NA + i * _BJ, _BJ), :],
            (((0,), (0,)), ((), ())), preferred_element_type=jnp.float32)
        part = acc[:, :_C] + acc[:, _C:]
        ut_ref[...] += part + jnp.where(i == 0, 1.0, 0.0) * ones_ref[...]

    @pl.when(jnp.logical_and(p == 2 * _STEPS - 1, i == _NB - 1))
    def _finish():
        out_a_ref[...] = ua_ref[...]
        out_t_ref[...] = ut_ref[...]


@functools.partial(jax.jit, static_argnames=())
def _mpnn(inputs, first_a, first_t, s2, s1):
    out_a, out_t = pl.pallas_call(
        _body,
        grid=(2 * _STEPS, _NB),
        in_specs=[
            pl.BlockSpec((_C, _C), lambda p, i: (0, 0)),
            pl.BlockSpec((_C, _C), lambda p, i: (0, 0)),
            pl.BlockSpec((_NA, _C), lambda p, i: (0, 0)),
            pl.BlockSpec((_NT, _C), lambda p, i: (0, 0)),
            pl.BlockSpec((_BJ, _NT),
                         lambda p, i: (jnp.where(p == 0, i, 0), 0)),
        ],
        out_specs=[
            pl.BlockSpec((_NA, _C), lambda p, i: (0, 0)),
            pl.BlockSpec((_NT, _C), lambda p, i: (0, 0)),
        ],
        out_shape=[
            jax.ShapeDtypeStruct((_NA, _C), jnp.float32),
            jax.ShapeDtypeStruct((_NT, _C), jnp.float32),
        ],
        scratch_shapes=[
            pltpu.VMEM((_NA, _C), jnp.float32),
            pltpu.VMEM((_NT, _C), jnp.float32),
            pltpu.VMEM((_NA, _NT), jnp.bfloat16),
            pltpu.VMEM((_NA, _NT), jnp.bfloat16),
            pltpu.VMEM((_NA, _NT), jnp.bfloat16),
            pltpu.VMEM((3 * _NT, 2 * _C), jnp.bfloat16),
            pltpu.VMEM((1, _C), jnp.float32),
        ],
        compiler_params=pltpu.CompilerParams(
            dimension_semantics=("arbitrary", "arbitrary")),
    )(s2, s1, first_a, first_t, inputs)
    return out_a, out_t


def kernel(inputs, first_a, first_t, Awij, Awij2):
    s2 = jnp.sum(Awij2, axis=1)
    s1 = jnp.sum(Awij, axis=1)
    out_a, out_t = _mpnn(inputs, first_a, first_t, s2, s1)
    pad = jnp.zeros((_NA, _C), dtype=out_a.dtype)
    top = jnp.concatenate([out_a, pad], axis=1)
    bot = jnp.concatenate([out_t, pad], axis=1)
    return jnp.concatenate([top, bot], axis=0)


# speedup vs baseline: 649.0352x; 1.0311x over previous
import functools

import jax
import jax.numpy as jnp
from jax.experimental import pallas as pl
from jax.experimental.pallas import tpu as pltpu

_NA, _NT = 2048, 2048
_C = 4
_STEPS = 2
_BJ = 1024
_NB = _NA // _BJ


def _split_hilo(us):
    uh = us.astype(jnp.bfloat16)
    ul = (us - uh.astype(jnp.float32)).astype(jnp.bfloat16)
    return jnp.concatenate([uh, ul], axis=1)


def _body(s2_ref, s1_ref, fa_ref, ft_ref, x_ref, out_a_ref, out_t_ref,
          ua_ref, ut_ref, m0_ref, m1_ref, m2_ref, rhs_ref, ones_ref):
    p = pl.program_id(0)
    i = pl.program_id(1)
    rows = pl.ds(i * _BJ, _BJ)

    @pl.when(jnp.logical_and(p == 0, i == 0))
    def _init():
        ua_ref[...] = fa_ref[...]
        ut_ref[...] = ft_ref[...]

    @pl.when(p == 0)
    def _build_masks():
        x = x_ref[...]
        m0_ref[rows, :] = (x == 0).astype(jnp.bfloat16)
        m1_ref[rows, :] = (x == 1).astype(jnp.bfloat16)
        m2_ref[rows, :] = (x == 2).astype(jnp.bfloat16)

    @pl.when(jnp.logical_and(p % 2 == 0, i == 0))
    def _rhs_a():
        s2 = s2_ref[...]
        r3 = s2[3:4, :]
        u = ut_ref[...]
        for e in range(3):
            rhs_ref[pl.ds(e * _NT, _NT), :] = _split_hilo(
                u * (s2[e:e + 1, :] - r3))
        ones_ref[...] = jnp.sum(u * r3, axis=0, keepdims=True)

    @pl.when(jnp.logical_and(p % 2 == 1, i == 0))
    def _rhs_t():
        s1 = s1_ref[...]
        r3 = s1[3:4, :]
        ua = ua_ref[...]
        for e in range(3):
            rhs_ref[pl.ds(e * _NA, _NA), :] = _split_hilo(
                ua * (s1[e:e + 1, :] - r3))
        ones_ref[...] = jnp.sum(ua * r3, axis=0, keepdims=True)

    @pl.when(p % 2 == 0)
    def _phase_a():
        acc = jax.lax.dot_general(
            m0_ref[rows, :], rhs_ref[pl.ds(0, _NT), :],
            (((1,), (0,)), ((), ())), preferred_element_type=jnp.float32)
        acc += jax.lax.dot_general(
            m1_ref[rows, :], rhs_ref[pl.ds(_NT, _NT), :],
            (((1,), (0,)), ((), ())), preferred_element_type=jnp.float32)
        acc += jax.lax.dot_general(
            m2_ref[rows, :], rhs_ref[pl.ds(2 * _NT, _NT), :],
            (((1,), (0,)), ((), ())), preferred_element_type=jnp.float32)
        msg = acc[:, :_C] + acc[:, _C:] + ones_ref[...]
        ua_ref[rows, :] += msg

    @pl.when(p % 2 == 1)
    def _phase_t():
        rr = pl.ds(i * _BJ, _BJ)
        acc = jax.lax.dot_general(
            m0_ref[rows, :], rhs_ref[rr, :],
            (((0,), (0,)), ((), ())), preferred_element_type=jnp.float32)
        acc += jax.lax.dot_general(
            m1_ref[rows, :], rhs_ref[pl.ds(_NA + i * _BJ, _BJ), :],
            (((0,), (0,)), ((), ())), preferred_element_type=jnp.float32)
        acc += jax.lax.dot_general(
            m2_ref[rows, :], rhs_ref[pl.ds(2 * _NA + i * _BJ, _BJ), :],
            (((0,), (0,)), ((), ())), preferred_element_type=jnp.float32)
        part = acc[:, :_C] + acc[:, _C:]
        ut_ref[...] += part + jnp.where(i == 0, 1.0, 0.0) * ones_ref[...]

    @pl.when(jnp.logical_and(p == 2 * _STEPS - 1, i == _NB - 1))
    def _finish():
        out_a_ref[...] = ua_ref[...]
        out_t_ref[...] = ut_ref[...]


@functools.partial(jax.jit, static_argnames=())
def _mpnn(inputs, first_a, first_t, s2, s1):
    out_a, out_t = pl.pallas_call(
        _body,
        grid=(2 * _STEPS, _NB),
        in_specs=[
            pl.BlockSpec((_C, _C), lambda p, i: (0, 0)),
            pl.BlockSpec((_C, _C), lambda p, i: (0, 0)),
            pl.BlockSpec((_NA, _C), lambda p, i: (0, 0)),
            pl.BlockSpec((_NT, _C), lambda p, i: (0, 0)),
            pl.BlockSpec((_BJ, _NT),
                         lambda p, i: (jnp.where(p == 0, i, 0), 0)),
        ],
        out_specs=[
            pl.BlockSpec((_NA, _C), lambda p, i: (0, 0)),
            pl.BlockSpec((_NT, _C), lambda p, i: (0, 0)),
        ],
        out_shape=[
            jax.ShapeDtypeStruct((_NA, _C), jnp.float32),
            jax.ShapeDtypeStruct((_NT, _C), jnp.float32),
        ],
        scratch_shapes=[
            pltpu.VMEM((_NA, _C), jnp.float32),
            pltpu.VMEM((_NT, _C), jnp.float32),
            pltpu.VMEM((_NA, _NT), jnp.bfloat16),
            pltpu.VMEM((_NA, _NT), jnp.bfloat16),
            pltpu.VMEM((_NA, _NT), jnp.bfloat16),
            pltpu.VMEM((3 * _NT, 2 * _C), jnp.bfloat16),
            pltpu.VMEM((1, _C), jnp.float32),
        ],
        compiler_params=pltpu.CompilerParams(
            dimension_semantics=("arbitrary", "arbitrary")),
    )(s2, s1, first_a, first_t, inputs)
    return out_a, out_t


def kernel(inputs, first_a, first_t, Awij, Awij2):
    s2 = jnp.sum(Awij2, axis=1)
    s1 = jnp.sum(Awij, axis=1)
    out_a, out_t = _mpnn(inputs, first_a, first_t, s2, s1)
    pad = jnp.zeros((_NA, _C), dtype=out_a.dtype)
    top = jnp.concatenate([out_a, pad], axis=1)
    bot = jnp.concatenate([out_t, pad], axis=1)
    return jnp.concatenate([top, bot], axis=0)
